# Initial kernel scaffold; baseline (speedup 1.0000x reference)
import jax
import jax.numpy as jnp
from jax.experimental import pallas as pl


def kernel(x, wSupp, chunk_map):
    raise NotImplementedError("write your pallas kernel here")


# speedup vs baseline: 5.4139x; 5.4139x over previous
import functools

import jax
import jax.numpy as jnp
from jax import lax
from jax.experimental import pallas as pl
from jax.experimental.pallas import tpu as pltpu
from jax.experimental.pallas import tpu_sc as plsc

_B = 1024
_NCLASS = 1000
_NSUPP = 64
_NCHUNK = 4096
_NCLS_PAD = 1024

_NC = 2
_NS = 16
_NW = _NC * _NS
_CLS_PER_W = _NCLS_PAD // _NW
_CLS_PER_ROUND = 16
_ROUNDS = _CLS_PER_W // _CLS_PER_ROUND


_BLK_WORDS = _NSUPP * _CLS_PER_ROUND


def _sc_build_w(cm_blk, w_blk):
    mesh = plsc.VectorSubcoreMesh(core_axis_name="c", subcore_axis_name="s")

    @functools.partial(
        pl.kernel,
        mesh=mesh,
        compiler_params=pltpu.CompilerParams(needs_layout_passes=False),
        out_type=jax.ShapeDtypeStruct((_NCLS_PAD * _NCHUNK,), jnp.float32),
        scratch_types=[
            pltpu.VMEM((_BLK_WORDS,), jnp.int32),
            pltpu.VMEM((_BLK_WORDS,), jnp.float32),
            pltpu.VMEM((_CLS_PER_ROUND * _NCHUNK,), jnp.float32),
        ],
    )
    def k(cm_hbm, w_hbm, wt_hbm, cm_v, w_v, buf):
        wid = lax.axis_index("s") * _NC + lax.axis_index("c")
        zv = jnp.zeros((16,), jnp.float32)
        row_base = lax.broadcasted_iota(jnp.int32, (16,), 0) * _NCHUNK
        for r in range(_ROUNDS):
            blk = wid * _ROUNDS + r
            pltpu.sync_copy(cm_hbm.at[pl.ds(blk * _BLK_WORDS, _BLK_WORDS)],
                            cm_v)
            pltpu.sync_copy(w_hbm.at[pl.ds(blk * _BLK_WORDS, _BLK_WORDS)],
                            w_v)

            def zero_body(i, carry):
                for u in range(8):
                    buf[pl.ds((i * 8 + u) * 16, 16)] = zv
                return carry

            lax.fori_loop(0, (_CLS_PER_ROUND * _NCHUNK) // (16 * 8),
                          zero_body, 0)

            for j in range(_NSUPP):
                idx = row_base + cm_v[pl.ds(j * 16, 16)]
                plsc.addupdate_scatter(buf, [idx], w_v[pl.ds(j * 16, 16)])

            pltpu.sync_copy(
                buf,
                wt_hbm.at[pl.ds(blk * _CLS_PER_ROUND * _NCHUNK,
                                _CLS_PER_ROUND * _NCHUNK)])

    return k(cm_blk, w_blk)


_BN = 256


def _tc_matmul_exp(x, wt):
    def body(x_ref, wt_ref, o_ref):
        acc = lax.dot_general(
            x_ref[...], wt_ref[...], (((1,), (1,)), ((), ())),
            preferred_element_type=jnp.float32)
        o_ref[...] = jnp.exp(acc)

    return pl.pallas_call(
        body,
        grid=(_NCLS_PAD // _BN,),
        in_specs=[
            pl.BlockSpec((_B, _NCHUNK), lambda j: (0, 0)),
            pl.BlockSpec((_BN, _NCHUNK), lambda j: (j, 0)),
        ],
        out_specs=pl.BlockSpec((_B, _BN), lambda j: (0, j)),
        out_shape=jax.ShapeDtypeStruct((_B, _NCLS_PAD), jnp.float32),
    )(x, wt)


def kernel(x, wSupp, chunk_map):
    pad = ((0, _NCLS_PAD - _NCLASS), (0, 0))
    cm_blk = (jnp.pad(chunk_map, pad)
              .reshape(_NCLS_PAD // _CLS_PER_ROUND, _CLS_PER_ROUND, _NSUPP)
              .transpose(0, 2, 1).reshape(-1))
    w_blk = (jnp.pad(wSupp, pad)
             .reshape(_NCLS_PAD // _CLS_PER_ROUND, _CLS_PER_ROUND, _NSUPP)
             .transpose(0, 2, 1).reshape(-1))
    wt = _sc_build_w(cm_blk, w_blk).reshape(_NCLS_PAD, _NCHUNK)
    out = _tc_matmul_exp(x, wt)
    return out[:, :_NCLASS]
